# baseline (speedup 1.0000x reference)
import functools

import jax
import jax.numpy as jnp
from jax import lax
from jax.experimental import pallas as pl
from jax.experimental.pallas import tpu as pltpu
from jax.experimental.pallas import tpu_sc as plsc

N_NODES = 10000
N_EDGES = 320000
D = 128

NC = 2
NS = 16
NW = NC * NS
EW = N_EDGES // NW
C = 80
EWP = -(-EW // C) * C
NCHUNK = EWP // C
ACC_ROWS = 10016
DUMMY = 10008
ROWS_PT = 624
TAIL_BASE = NS * ROWS_PT
TAIL_Z = ACC_ROWS - TAIL_BASE
TAIL_WB = N_NODES - TAIL_BASE
ZR = 104
ZC = 2504
WBC = 2000


def _sc_scatter_mean(x, eidx):
    mesh = plsc.VectorSubcoreMesh(core_axis_name="c", subcore_axis_name="s")

    @functools.partial(
        pl.kernel,
        out_type=[
            jax.ShapeDtypeStruct((NC * N_NODES, D), jnp.float32),
            jax.ShapeDtypeStruct((NC * N_NODES,), jnp.float32),
        ],
        mesh=mesh,
        scratch_types=[
            pltpu.VMEM_SHARED((ACC_ROWS, D), jnp.float32),
            pltpu.VMEM_SHARED((ACC_ROWS,), jnp.float32),
            pltpu.VMEM((ZR, D), jnp.float32),
            pltpu.VMEM((C, D), jnp.float32),
            pltpu.VMEM((C, D), jnp.float32),
            pltpu.VMEM((2, C), jnp.int32),
            pltpu.VMEM((2, C), jnp.int32),
            pltpu.VMEM((C,), jnp.float32),
            pltpu.VMEM((ZC,), jnp.float32),
            pltpu.SemaphoreType.DMA,
            pltpu.SemaphoreType.DMA,
            pltpu.SemaphoreType.DMA,
            pltpu.SemaphoreType.DMA,
            pltpu.SemaphoreType.DMA,
            pltpu.SemaphoreType.DMA,
            pltpu.SemaphoreType.DMA,
        ],
    )
    def k(x_hbm, e_hbm, part_hbm, cnt_hbm,
          acc, cnt, zrows, rows0, rows1, ib0, ib1, ones, zvec,
          sem_i0, sem_i1, sem0, sem1, sem_s0, sem_s1, sem_z):
        cid = lax.axis_index("c")
        sid = lax.axis_index("s")
        wid = sid * NC + cid

        zero16 = jnp.zeros((16,), jnp.float32)
        one16 = jnp.full((16,), 1.0, jnp.float32)

        pltpu.async_copy(e_hbm.at[wid, 0], ib0, sem_i0)
        pltpu.async_copy(e_hbm.at[wid, 1], ib1, sem_i1)

        @pl.loop(0, ZR)
        def _(r):
            @pl.loop(0, D, step=16)
            def _(j):
                zrows[r, pl.ds(j, 16)] = zero16

        @pl.loop(0, C, step=16)
        def _(j):
            ones[pl.ds(j, 16)] = one16

        base_r = sid * ROWS_PT

        @pl.loop(0, ROWS_PT, step=ZR)
        def _(r):
            pltpu.async_copy(zrows, acc.at[pl.ds(base_r + r, ZR)], sem_z)

        @pl.when(sid == NS - 1)
        def _():
            pltpu.async_copy(zrows.at[pl.ds(0, TAIL_Z)],
                             acc.at[pl.ds(TAIL_BASE, TAIL_Z)], sem_z)

        @pl.when(sid == 0)
        def _():
            @pl.loop(0, ZC, step=16)
            def _(j):
                zvec[pl.ds(j, 16)] = zero16

            @pl.loop(0, ACC_ROWS, step=ZC)
            def _(j):
                pltpu.async_copy(zvec, cnt.at[pl.ds(j, ZC)], sem_z)

            @pl.loop(0, ACC_ROWS, step=ZC)
            def _(j):
                pltpu.make_async_copy(zvec, cnt.at[pl.ds(j, ZC)], sem_z).wait()

        pltpu.make_async_copy(e_hbm.at[wid, 0], ib0, sem_i0).wait()
        pltpu.async_copy(x_hbm.at[ib0.at[0]], rows0, sem0)

        @pl.loop(0, ROWS_PT, step=ZR)
        def _(r):
            pltpu.make_async_copy(zrows, acc.at[pl.ds(base_r + r, ZR)], sem_z).wait()

        @pl.when(sid == NS - 1)
        def _():
            pltpu.make_async_copy(zrows.at[pl.ds(0, TAIL_Z)],
                                  acc.at[pl.ds(TAIL_BASE, TAIL_Z)], sem_z).wait()

        plsc.subcore_barrier()

        @pl.loop(0, NCHUNK - 1, step=2)
        def _(i):
            pltpu.make_async_copy(e_hbm.at[wid, i + 1], ib1, sem_i1).wait()
            pltpu.async_copy(x_hbm.at[ib1.at[0]], rows1, sem1)
            pltpu.make_async_copy(x_hbm.at[pl.ds(0, C)], rows0, sem0).wait()
            pltpu.async_copy(rows0, acc.at[ib0.at[1]], sem_s0, add=True)
            pltpu.sync_copy(ones, cnt.at[ib0.at[1]], add=True)
            pltpu.make_async_copy(x_hbm.at[pl.ds(0, C)], rows1, sem1).wait()
            pltpu.async_copy(rows1, acc.at[ib1.at[1]], sem_s1, add=True)
            pltpu.sync_copy(ones, cnt.at[ib1.at[1]], add=True)
            pltpu.make_async_copy(rows0, acc.at[ib0.at[1]], sem_s0).wait()
            pltpu.async_copy(e_hbm.at[wid, i + 2], ib0, sem_i0)
            pltpu.make_async_copy(e_hbm.at[wid, i + 2], ib0, sem_i0).wait()
            pltpu.async_copy(x_hbm.at[ib0.at[0]], rows0, sem0)
            pltpu.make_async_copy(rows1, acc.at[ib1.at[1]], sem_s1).wait()

            @pl.when(i + 3 < NCHUNK)
            def _():
                pltpu.async_copy(e_hbm.at[wid, i + 3], ib1, sem_i1)

        pltpu.make_async_copy(x_hbm.at[pl.ds(0, C)], rows0, sem0).wait()
        pltpu.sync_copy(rows0, acc.at[ib0.at[1]], add=True)
        pltpu.sync_copy(ones, cnt.at[ib0.at[1]], add=True)

        plsc.subcore_barrier()

        obase = cid * N_NODES
        pltpu.sync_copy(acc.at[pl.ds(base_r, ROWS_PT)],
                        part_hbm.at[pl.ds(obase + base_r, ROWS_PT)])

        @pl.when(sid == NS - 1)
        def _():
            pltpu.sync_copy(acc.at[pl.ds(TAIL_BASE, TAIL_WB)],
                            part_hbm.at[pl.ds(obase + TAIL_BASE, TAIL_WB)])

        @pl.when(sid == 0)
        def _():
            @pl.loop(0, N_NODES, step=WBC)
            def _(j):
                pltpu.sync_copy(cnt.at[pl.ds(j, WBC)], zvec.at[pl.ds(0, WBC)])
                pltpu.sync_copy(zvec.at[pl.ds(0, WBC)],
                                cnt_hbm.at[pl.ds(obase + j, WBC)])

    return k(x, eidx)


BN = 1000


def _tc_combine(part, cnt3, W, b2):
    def body(p_ref, c_ref, w_ref, b_ref, o_ref):
        acc = p_ref[0] + p_ref[1]
        c = c_ref[0] + c_ref[1]
        scale = 1.0 / jnp.maximum(c, 1.0)
        ind = jnp.minimum(c, 1.0)
        y = lax.dot_general(acc * scale, w_ref[...],
                            (((1,), (1,)), ((), ())),
                            preferred_element_type=jnp.float32)
        o_ref[...] = y + b_ref[...] * ind

    return pl.pallas_call(
        body,
        grid=(N_NODES // BN,),
        in_specs=[
            pl.BlockSpec((NC, BN, D), lambda i: (0, i, 0)),
            pl.BlockSpec((NC, BN, 1), lambda i: (0, i, 0)),
            pl.BlockSpec((D, D), lambda i: (0, 0)),
            pl.BlockSpec((1, D), lambda i: (0, 0)),
        ],
        out_specs=pl.BlockSpec((BN, D), lambda i: (i, 0)),
        out_shape=jax.ShapeDtypeStruct((N_NODES, D), jnp.float32),
    )(part, cnt3, W, b2)


def kernel(x, edge_index, W, b):
    if EWP == EW:
        eidx = edge_index.reshape(2, NW, NCHUNK, C).transpose(1, 2, 0, 3)
    else:
        srcp = jnp.pad(edge_index[0].reshape(NW, EW), ((0, 0), (0, EWP - EW)))
        dstp = jnp.pad(edge_index[1].reshape(NW, EW), ((0, 0), (0, EWP - EW)),
                       constant_values=DUMMY)
        trash = (N_NODES
                 + (jnp.arange(NW, dtype=jnp.int32) % (ACC_ROWS - N_NODES)))
        dstp = jnp.where(jnp.arange(EWP)[None, :] < EW, dstp, trash[:, None])
        eidx = jnp.stack([srcp, dstp], axis=1).reshape(NW, 2, NCHUNK, C)
        eidx = eidx.transpose(0, 2, 1, 3)
    part, cnt = _sc_scatter_mean(x, eidx)
    part = part.reshape(NC, N_NODES, D)
    cnt3 = cnt.reshape(NC, N_NODES, 1)
    return _tc_combine(part, cnt3, W, b.reshape(1, D))


# speedup vs baseline: 1.0482x; 1.0482x over previous
import functools

import jax
import jax.numpy as jnp
from jax import lax
from jax.experimental import pallas as pl
from jax.experimental.pallas import tpu as pltpu
from jax.experimental.pallas import tpu_sc as plsc

N_NODES = 10000
N_EDGES = 320000
D = 128

NC = 2
NS = 16
NW = NC * NS
EW = N_EDGES // NW
C = 80
EWP = -(-EW // C) * C
NCHUNK = EWP // C
ACC_ROWS = 10016
DUMMY = 10008
ROWS_PT = 624
TAIL_BASE = NS * ROWS_PT
TAIL_Z = ACC_ROWS - TAIL_BASE
TAIL_WB = N_NODES - TAIL_BASE
ZR = 104
ZC = 2504
WBC = 2000


def _sc_scatter_mean(x, eidx):
    mesh = plsc.VectorSubcoreMesh(core_axis_name="c", subcore_axis_name="s")

    @functools.partial(
        pl.kernel,
        out_type=[
            jax.ShapeDtypeStruct((NC * N_NODES, D), jnp.float32),
            jax.ShapeDtypeStruct((NC * N_NODES,), jnp.float32),
        ],
        mesh=mesh,
        scratch_types=[
            pltpu.VMEM_SHARED((ACC_ROWS, D), jnp.float32),
            pltpu.VMEM_SHARED((ACC_ROWS,), jnp.float32),
            pltpu.VMEM((ZR, D), jnp.float32),
            pltpu.VMEM((C, D), jnp.float32),
            pltpu.VMEM((C, D), jnp.float32),
            pltpu.VMEM((C,), jnp.int32),
            pltpu.VMEM((C,), jnp.int32),
            pltpu.VMEM((C,), jnp.int32),
            pltpu.VMEM((C,), jnp.int32),
            pltpu.VMEM((C,), jnp.float32),
            pltpu.VMEM((ZC,), jnp.float32),
            pltpu.SemaphoreType.DMA,
            pltpu.SemaphoreType.DMA,
            pltpu.SemaphoreType.DMA,
            pltpu.SemaphoreType.DMA,
            pltpu.SemaphoreType.DMA,
        ],
    )
    def k(x_hbm, e_hbm, part_hbm, cnt_hbm,
          acc, cnt, zrows, rows0, rows1, is0, id0, is1, id1, ones, zvec,
          sem_i0, sem_i1, sem0, sem1, sem_z):
        cid = lax.axis_index("c")
        sid = lax.axis_index("s")
        wid = sid * NC + cid

        zero16 = jnp.zeros((16,), jnp.float32)
        one16 = jnp.full((16,), 1.0, jnp.float32)

        def idx_start(j, isb, idb, sem):
            pltpu.async_copy(e_hbm.at[0, wid, j], isb, sem)
            pltpu.async_copy(e_hbm.at[1, wid, j], idb, sem)

        def idx_wait(j, isb, idb, sem):
            pltpu.make_async_copy(e_hbm.at[0, wid, j], isb, sem).wait()
            pltpu.make_async_copy(e_hbm.at[1, wid, j], idb, sem).wait()

        idx_start(0, is0, id0, sem_i0)
        idx_start(1, is1, id1, sem_i1)

        @pl.loop(0, ZR)
        def _(r):
            @pl.loop(0, D, step=16)
            def _(j):
                zrows[r, pl.ds(j, 16)] = zero16

        @pl.loop(0, C, step=16)
        def _(j):
            ones[pl.ds(j, 16)] = one16

        base_r = sid * ROWS_PT

        @pl.loop(0, ROWS_PT, step=ZR)
        def _(r):
            pltpu.async_copy(zrows, acc.at[pl.ds(base_r + r, ZR)], sem_z)

        @pl.when(sid == NS - 1)
        def _():
            pltpu.async_copy(zrows.at[pl.ds(0, TAIL_Z)],
                             acc.at[pl.ds(TAIL_BASE, TAIL_Z)], sem_z)

        @pl.when(sid == 0)
        def _():
            @pl.loop(0, ZC, step=16)
            def _(j):
                zvec[pl.ds(j, 16)] = zero16

            @pl.loop(0, ACC_ROWS, step=ZC)
            def _(j):
                pltpu.async_copy(zvec, cnt.at[pl.ds(j, ZC)], sem_z)

            @pl.loop(0, ACC_ROWS, step=ZC)
            def _(j):
                pltpu.make_async_copy(zvec, cnt.at[pl.ds(j, ZC)], sem_z).wait()

        idx_wait(0, is0, id0, sem_i0)
        pltpu.async_copy(x_hbm.at[is0], rows0, sem0)

        @pl.loop(0, ROWS_PT, step=ZR)
        def _(r):
            pltpu.make_async_copy(zrows, acc.at[pl.ds(base_r + r, ZR)], sem_z).wait()

        @pl.when(sid == NS - 1)
        def _():
            pltpu.make_async_copy(zrows.at[pl.ds(0, TAIL_Z)],
                                  acc.at[pl.ds(TAIL_BASE, TAIL_Z)], sem_z).wait()

        plsc.subcore_barrier()

        @pl.loop(0, NCHUNK - 1, step=2)
        def _(i):
            idx_wait(i + 1, is1, id1, sem_i1)
            pltpu.async_copy(x_hbm.at[is1], rows1, sem1)
            pltpu.make_async_copy(x_hbm.at[pl.ds(0, C)], rows0, sem0).wait()
            pltpu.sync_copy(rows0, acc.at[id0], add=True)
            pltpu.sync_copy(ones, cnt.at[id0], add=True)
            idx_start(i + 2, is0, id0, sem_i0)
            idx_wait(i + 2, is0, id0, sem_i0)
            pltpu.async_copy(x_hbm.at[is0], rows0, sem0)
            pltpu.make_async_copy(x_hbm.at[pl.ds(0, C)], rows1, sem1).wait()
            pltpu.sync_copy(rows1, acc.at[id1], add=True)
            pltpu.sync_copy(ones, cnt.at[id1], add=True)
            @pl.when(i + 3 < NCHUNK)
            def _():
                idx_start(i + 3, is1, id1, sem_i1)

        pltpu.make_async_copy(x_hbm.at[pl.ds(0, C)], rows0, sem0).wait()
        pltpu.sync_copy(rows0, acc.at[id0], add=True)
        pltpu.sync_copy(ones, cnt.at[id0], add=True)

        plsc.subcore_barrier()

        obase = cid * N_NODES
        pltpu.sync_copy(acc.at[pl.ds(base_r, ROWS_PT)],
                        part_hbm.at[pl.ds(obase + base_r, ROWS_PT)])

        @pl.when(sid == NS - 1)
        def _():
            pltpu.sync_copy(acc.at[pl.ds(TAIL_BASE, TAIL_WB)],
                            part_hbm.at[pl.ds(obase + TAIL_BASE, TAIL_WB)])

        @pl.when(sid == 0)
        def _():
            @pl.loop(0, N_NODES, step=WBC)
            def _(j):
                pltpu.sync_copy(cnt.at[pl.ds(j, WBC)], zvec.at[pl.ds(0, WBC)])
                pltpu.sync_copy(zvec.at[pl.ds(0, WBC)],
                                cnt_hbm.at[pl.ds(obase + j, WBC)])

    return k(x, eidx)


BN = 1000


def _tc_combine(part, cnt3, W, b2):
    def body(p_ref, c_ref, w_ref, b_ref, o_ref):
        acc = p_ref[0] + p_ref[1]
        c = c_ref[0] + c_ref[1]
        scale = 1.0 / jnp.maximum(c, 1.0)
        ind = jnp.minimum(c, 1.0)
        y = lax.dot_general(acc * scale, w_ref[...],
                            (((1,), (1,)), ((), ())),
                            preferred_element_type=jnp.float32)
        o_ref[...] = y + b_ref[...] * ind

    return pl.pallas_call(
        body,
        grid=(N_NODES // BN,),
        in_specs=[
            pl.BlockSpec((NC, BN, D), lambda i: (0, i, 0)),
            pl.BlockSpec((NC, BN, 1), lambda i: (0, i, 0)),
            pl.BlockSpec((D, D), lambda i: (0, 0)),
            pl.BlockSpec((1, D), lambda i: (0, 0)),
        ],
        out_specs=pl.BlockSpec((BN, D), lambda i: (i, 0)),
        out_shape=jax.ShapeDtypeStruct((N_NODES, D), jnp.float32),
    )(part, cnt3, W, b2)


def kernel(x, edge_index, W, b):
    if EWP == EW:
        eidx = edge_index.reshape(2, NW, NCHUNK, C)
    else:
        srcp = jnp.pad(edge_index[0].reshape(NW, EW), ((0, 0), (0, EWP - EW)))
        dstp = jnp.pad(edge_index[1].reshape(NW, EW), ((0, 0), (0, EWP - EW)),
                       constant_values=DUMMY)
        trash = (N_NODES
                 + (jnp.arange(NW, dtype=jnp.int32) % (ACC_ROWS - N_NODES)))
        dstp = jnp.where(jnp.arange(EWP)[None, :] < EW, dstp, trash[:, None])
        eidx = jnp.stack([srcp, dstp], axis=0).reshape(2, NW, NCHUNK, C)
    part, cnt = _sc_scatter_mean(x, eidx)
    part = part.reshape(NC, N_NODES, D)
    cnt3 = cnt.reshape(NC, N_NODES, 1)
    return _tc_combine(part, cnt3, W, b.reshape(1, D))
